# baseline (speedup 1.0000x reference)
import functools

import jax
import jax.numpy as jnp
from jax import lax
from jax.experimental import pallas as pl
from jax.experimental.pallas import tpu as pltpu
from jax.experimental.pallas import tpu_sc as plsc


def _make_sc_transpose(V, D):
    info = plsc.get_sparse_core_info()
    NC, NS = info.num_cores, info.num_subcores
    NW = NC * NS
    D2 = 2 * D
    nt_full = V // 128
    rem = V - nt_full * 128
    assert rem % 2 == 0

    mesh = plsc.VectorSubcoreMesh(core_axis_name="c", subcore_axis_name="s")

    @functools.partial(
        pl.kernel,
        mesh=mesh,
        compiler_params=pltpu.CompilerParams(use_tc_tiling_on_sc=True,
                                             needs_layout_passes=False),
        out_type=[
            jax.ShapeDtypeStruct((V // 2, D2), jnp.float32),
            jax.ShapeDtypeStruct((V // 2, D2), jnp.float32),
            jax.ShapeDtypeStruct((V // 2, D2), jnp.float32),
        ],
        scratch_types=[
            pltpu.VMEM((D, 128), jnp.float32),
            pltpu.VMEM((64, D2), jnp.float32),
        ],
    )
    def sc_transpose(umem_t, uout_t, imem_t, edge_u, edge_o, edge_i,
                     umem2, uout2, imem2, in_blk, out_blk):
        wid = lax.axis_index("s") * NC + lax.axis_index("c")
        iota = lax.iota(jnp.int32, 16)

        def do_tile(src, dst, t):
            col0 = pl.multiple_of(t * 128, 128)
            pltpu.sync_copy(src.at[:, pl.ds(col0, 128)], in_blk)
            for p in range(64):
                for par in range(2):
                    c = 2 * p + par
                    for g in range(D // 16):
                        v = plsc.load_gather(
                            in_blk, [iota + g * 16, jnp.full((16,), c, jnp.int32)])
                        out_blk[p, pl.ds(par * D + g * 16, 16)] = v
            row0 = pl.multiple_of(t * 64, 64)
            pltpu.sync_copy(out_blk, dst.at[pl.ds(row0, 64)])

        for src, dst in ((umem_t, umem2), (uout_t, uout2), (imem_t, imem2)):
            def body(k, carry):
                do_tile(src, dst, wid + k * NW)
                return carry
            n_t = (nt_full - wid + NW - 1) // NW
            lax.fori_loop(0, n_t, body, 0)

        if rem:
            nr = rem // 2
            @pl.when(wid == nt_full % NW)
            def _():
                for edge, dst in ((edge_u, umem2), (edge_o, uout2),
                                  (edge_i, imem2)):
                    pltpu.sync_copy(edge, out_blk.at[pl.ds(0, nr)])
                    pltpu.sync_copy(out_blk.at[pl.ds(0, nr)],
                                    dst.at[pl.ds(V // 2 - nr, nr)])

    return sc_transpose


def _make_sc_gather(n_main, n_cur, D2):
    info = plsc.get_sparse_core_info()
    NC, NS = info.num_cores, info.num_subcores
    NW = NC * NS
    assert n_main % NW == 0 and n_cur % NW == 0
    pw = n_main // NW
    C = 256
    assert pw % C == 0
    nchunk = pw // C
    piw = n_cur // NW
    assert piw % 8 == 0

    mesh = plsc.VectorSubcoreMesh(core_axis_name="c", subcore_axis_name="s")

    @functools.partial(
        pl.kernel,
        mesh=mesh,
        compiler_params=pltpu.CompilerParams(use_tc_tiling_on_sc=True),
        out_type=[
            jax.ShapeDtypeStruct((n_main, D2), jnp.float32),
            jax.ShapeDtypeStruct((n_main, D2), jnp.float32),
            jax.ShapeDtypeStruct((n_cur, D2), jnp.float32),
            jax.ShapeDtypeStruct((n_cur, D2), jnp.float32),
        ],
        scratch_types=[
            pltpu.VMEM((C,), jnp.int32),
            pltpu.VMEM((C, D2), jnp.float32),
            pltpu.VMEM((C, D2), jnp.float32),
            pltpu.VMEM((piw,), jnp.int32),
            pltpu.VMEM((piw,), jnp.int32),
            pltpu.VMEM((piw, D2), jnp.float32),
            pltpu.VMEM((piw, D2), jnp.float32),
            pltpu.SemaphoreType.DMA,
            pltpu.SemaphoreType.DMA,
        ],
    )
    def sc_gather(idx_hbm, umem_hbm, uout_hbm, uidx_hbm, iidx_hbm, imem_hbm,
                  bmem_hbm, bout_hbm, su_hbm, si_hbm,
                  idx_v, rows_a, rows_b, uidx_v, iidx_v, crows_a, crows_b,
                  sem_a, sem_b):
        wid = lax.axis_index("s") * NC + lax.axis_index("c")
        base = wid * pw

        def chunk(c, carry):
            off = base + c * C
            pltpu.sync_copy(idx_hbm.at[pl.ds(off, C)], idx_v)
            cp_a = pltpu.async_copy(umem_hbm.at[idx_v], rows_a, sem_a)
            cp_b = pltpu.async_copy(uout_hbm.at[idx_v], rows_b, sem_b)
            cp_a.wait()
            pltpu.sync_copy(rows_a, bmem_hbm.at[pl.ds(off, C)])
            cp_b.wait()
            pltpu.sync_copy(rows_b, bout_hbm.at[pl.ds(off, C)])
            return carry

        lax.fori_loop(0, nchunk, chunk, 0)

        ibase = wid * piw
        pltpu.sync_copy(uidx_hbm.at[pl.ds(ibase, piw)], uidx_v)
        pltpu.sync_copy(iidx_hbm.at[pl.ds(ibase, piw)], iidx_v)
        cp_a = pltpu.async_copy(umem_hbm.at[uidx_v], crows_a, sem_a)
        cp_b = pltpu.async_copy(imem_hbm.at[iidx_v], crows_b, sem_b)
        cp_a.wait()
        pltpu.sync_copy(crows_a, su_hbm.at[pl.ds(ibase, piw)])
        cp_b.wait()
        pltpu.sync_copy(crows_b, si_hbm.at[pl.ds(ibase, piw)])

    return sc_gather


def _dense_body(nb_ref, ub_ref, ib_ref, bmem_ref, bout_ref, su_ref, si_ref,
                whop_ref, bhop_ref, wout_ref, bout2_ref, w1_ref, score_ref):
    d = whop_ref.shape[0]

    def half(packed, idx):
        sel = (idx & 1) == 1
        return jnp.where(sel, packed[..., d:], packed[..., :d])

    mem = half(bmem_ref[...], nb_ref[...])
    nout = half(bout_ref[...], nb_ref[...])
    cu = half(su_ref[...], ub_ref[...])
    ci = half(si_ref[...], ib_ref[...])
    q = cu + ci

    def attend(query):
        s = jnp.sum(query[:, None, :] * mem, axis=2)
        s = s - jnp.max(s, axis=-1, keepdims=True)
        e = jnp.exp(s)
        a = e / jnp.sum(e, axis=-1, keepdims=True)
        return jnp.sum(a[:, :, None] * nout, axis=1)

    w0 = attend(q)
    q1 = jnp.dot(q, whop_ref[...], preferred_element_type=jnp.float32)
    q1 = jnp.maximum(q1 + bhop_ref[...] + w0, 0.0)
    w1 = attend(q1)

    aa = cu * ci
    z = (jnp.dot(aa, wout_ref[:d], preferred_element_type=jnp.float32)
         + jnp.dot(w1, wout_ref[d:], preferred_element_type=jnp.float32)
         + bout2_ref[...])
    score = jnp.maximum(jnp.sum(z * w1_ref[...], axis=1, keepdims=True), 0.0)
    score_ref[...] = score


def _dense(nb, ub, ib, bmem, bout, su, si, W_hop, b_hop, W_out, b_out, W_1,
           interpret=False):
    B = su.shape[0]
    L = bmem.shape[1]
    D = W_hop.shape[0]
    BB = 128
    grid = (B // BB,)
    return pl.pallas_call(
        _dense_body,
        grid=grid,
        in_specs=[
            pl.BlockSpec((BB, L, 1), lambda i: (i, 0, 0)),
            pl.BlockSpec((BB, 1), lambda i: (i, 0)),
            pl.BlockSpec((BB, 1), lambda i: (i, 0)),
            pl.BlockSpec((BB, L, 2 * D), lambda i: (i, 0, 0)),
            pl.BlockSpec((BB, L, 2 * D), lambda i: (i, 0, 0)),
            pl.BlockSpec((BB, 2 * D), lambda i: (i, 0)),
            pl.BlockSpec((BB, 2 * D), lambda i: (i, 0)),
            pl.BlockSpec((D, D), lambda i: (0, 0)),
            pl.BlockSpec((1, D), lambda i: (0, 0)),
            pl.BlockSpec((2 * D, D), lambda i: (0, 0)),
            pl.BlockSpec((1, D), lambda i: (0, 0)),
            pl.BlockSpec((1, D), lambda i: (0, 0)),
        ],
        out_specs=pl.BlockSpec((BB, 1), lambda i: (i, 0)),
        out_shape=jax.ShapeDtypeStruct((B, 1), jnp.float32),
        interpret=interpret,
    )(nb, ub, ib, bmem, bout, su, si, W_hop, b_hop, W_out, b_out, W_1)


def kernel(input_users, input_items, input_items_negative, input_neighborhoods,
           input_neighborhood_lengths, input_neighborhoods_negative,
           input_neighborhood_lengths_negative, user_memory, user_output,
           item_memory, W_hop, b_hop, W_out, b_out, W_1):
    B, L = input_neighborhoods.shape
    V, D = user_memory.shape

    nb = input_neighborhoods.astype(jnp.int32)
    ub = input_users.astype(jnp.int32).reshape(B, 1)
    ib = input_items.astype(jnp.int32).reshape(B, 1)

    sc_transpose = _make_sc_transpose(V, D)
    rem = V % 128
    nr = rem // 2
    umem2, uout2, imem2 = sc_transpose(
        user_memory.T, user_output.T, item_memory.T,
        user_memory[V - rem:].reshape(nr, 2 * D),
        user_output[V - rem:].reshape(nr, 2 * D),
        item_memory[V - rem:].reshape(nr, 2 * D))

    sc_gather = _make_sc_gather(B * L, B, 2 * D)
    bmem, bout, su, si = sc_gather(
        jnp.right_shift(nb, 1).reshape(-1), umem2, uout2,
        jnp.right_shift(ub, 1).reshape(-1), jnp.right_shift(ib, 1).reshape(-1),
        imem2)

    return _dense(nb.reshape(B, L, 1), ub, ib,
                  bmem.reshape(B, L, 2 * D), bout.reshape(B, L, 2 * D),
                  su, si, W_hop,
                  b_hop.reshape(1, D), W_out, b_out.reshape(1, D),
                  W_1.reshape(1, D))


# speedup vs baseline: 1.3707x; 1.3707x over previous
import functools

import jax
import jax.numpy as jnp
from jax import lax
from jax.experimental import pallas as pl
from jax.experimental.pallas import tpu as pltpu
from jax.experimental.pallas import tpu_sc as plsc


def _make_sc_transpose(V, D, interpret=False):
    info = plsc.get_sparse_core_info()
    NC, NS = info.num_cores, info.num_subcores
    NW = NC * NS
    D2 = 2 * D
    npair = V // 128
    rem = V - npair * 128
    assert rem % 2 == 0

    mesh = plsc.VectorSubcoreMesh(core_axis_name="c", subcore_axis_name="s")

    @functools.partial(
        pl.kernel,
        mesh=mesh,
        interpret=interpret,
        compiler_params=pltpu.CompilerParams(use_tc_tiling_on_sc=True,
                                             needs_layout_passes=False),
        out_type=[
            jax.ShapeDtypeStruct((V // 2, D2), jnp.float32),
            jax.ShapeDtypeStruct((V // 2, D2), jnp.float32),
            jax.ShapeDtypeStruct((V // 2, D2), jnp.float32),
        ],
        scratch_types=[
            pltpu.VMEM((D, 128), jnp.float32),
            pltpu.VMEM((D, 128), jnp.float32),
            pltpu.VMEM((64, D2), jnp.float32),
            pltpu.VMEM((64, D2), jnp.float32),
            pltpu.SemaphoreType.DMA,
            pltpu.SemaphoreType.DMA,
            pltpu.SemaphoreType.DMA,
            pltpu.SemaphoreType.DMA,
        ],
    )
    def sc_transpose(umem_t, uout_t, imem_t, edge_u, edge_o, edge_i,
                     umem2, uout2, imem2,
                     in_a, in_b, out_a, out_b, sem_ia, sem_ib, sem_oa, sem_ob):
        wid = lax.axis_index("s") * NC + lax.axis_index("c")
        iota = lax.iota(jnp.int32, 16)
        ibufs = ((in_a, sem_ia), (in_b, sem_ib))
        obufs = ((out_a, sem_oa), (out_b, sem_ob))

        def src_cols(src, t):
            return src.at[:, pl.ds(pl.multiple_of(t * 128, 128), 128)]

        def dst_rows(dst, t):
            return dst.at[pl.ds(pl.multiple_of(t * 64, 64), 64)]

        def compute(inb, outb):
            def prow(p0, carry):
                for pp in range(4):
                    p = p0 * 4 + pp
                    pvec = jnp.full((16,), p, jnp.int32)
                    for par in range(2):
                        cvec = jnp.full((16,), 2 * p + par, jnp.int32)
                        for g in range(D // 16):
                            v = plsc.load_gather(inb, [iota + g * 16, cvec])
                            plsc.store_scatter(
                                outb, [pvec, iota + (par * D + g * 16)], v)
                return carry
            lax.fori_loop(0, 16, prow, 0)

        def run_table(src, dst):
            n_my = (npair - wid + NW - 1) // NW

            @pl.when(n_my > 0)
            def _():
                pltpu.async_copy(src_cols(src, wid), in_a, sem_ia)

            @pl.when(n_my > 1)
            def _():
                pltpu.async_copy(src_cols(src, wid + NW), in_b, sem_ib)

            def body2(k2, carry):
                for ph in range(2):
                    inb, sin = ibufs[ph]
                    outb, sout = obufs[ph]
                    k = k2 * 2 + ph

                    @pl.when(k < n_my)
                    def _():
                        t = wid + k * NW
                        pltpu.make_async_copy(src_cols(src, t), inb, sin).wait()

                        @pl.when(k >= 2)
                        def _():
                            prev = wid + (k - 2) * NW
                            pltpu.make_async_copy(
                                outb, dst_rows(dst, prev), sout).wait()

                        compute(inb, outb)
                        pltpu.async_copy(outb, dst_rows(dst, t), sout)

                        @pl.when(k + 2 < n_my)
                        def _():
                            nxt = wid + (k + 2) * NW
                            pltpu.async_copy(src_cols(src, nxt), inb, sin)
                return carry

            lax.fori_loop(0, (n_my + 1) // 2, body2, 0)

            for ph in range(2):
                outb, sout = obufs[ph]
                for back in (1, 2):
                    @pl.when((n_my >= back) & ((n_my - back) % 2 == ph))
                    def _():
                        t = wid + (n_my - back) * NW
                        pltpu.make_async_copy(outb, dst_rows(dst, t),
                                              sout).wait()

        for src, dst in ((umem_t, umem2), (uout_t, uout2), (imem_t, imem2)):
            run_table(src, dst)

        if rem:
            nr = rem // 2
            @pl.when(wid == 0)
            def _():
                for edge, dst in ((edge_u, umem2), (edge_o, uout2),
                                  (edge_i, imem2)):
                    pltpu.sync_copy(edge, out_a.at[pl.ds(0, nr)])
                    pltpu.sync_copy(out_a.at[pl.ds(0, nr)],
                                    dst.at[pl.ds(V // 2 - nr, nr)])

    return sc_transpose


def _make_sc_gather(n_main, n_cur, D2):
    info = plsc.get_sparse_core_info()
    NC, NS = info.num_cores, info.num_subcores
    NW = NC * NS
    assert n_main % NW == 0 and n_cur % NW == 0
    pw = n_main // NW
    C = 256
    assert pw % C == 0
    nchunk = pw // C
    piw = n_cur // NW
    assert piw % 8 == 0

    mesh = plsc.VectorSubcoreMesh(core_axis_name="c", subcore_axis_name="s")

    @functools.partial(
        pl.kernel,
        mesh=mesh,
        compiler_params=pltpu.CompilerParams(use_tc_tiling_on_sc=True),
        out_type=[
            jax.ShapeDtypeStruct((n_main, D2), jnp.float32),
            jax.ShapeDtypeStruct((n_main, D2), jnp.float32),
            jax.ShapeDtypeStruct((n_cur, D2), jnp.float32),
            jax.ShapeDtypeStruct((n_cur, D2), jnp.float32),
        ],
        scratch_types=[
            pltpu.VMEM((C,), jnp.int32),
            pltpu.VMEM((C, D2), jnp.float32),
            pltpu.VMEM((C, D2), jnp.float32),
            pltpu.VMEM((piw,), jnp.int32),
            pltpu.VMEM((piw,), jnp.int32),
            pltpu.VMEM((piw, D2), jnp.float32),
            pltpu.VMEM((piw, D2), jnp.float32),
            pltpu.SemaphoreType.DMA,
            pltpu.SemaphoreType.DMA,
        ],
    )
    def sc_gather(idx_hbm, umem_hbm, uout_hbm, uidx_hbm, iidx_hbm, imem_hbm,
                  bmem_hbm, bout_hbm, su_hbm, si_hbm,
                  idx_v, rows_a, rows_b, uidx_v, iidx_v, crows_a, crows_b,
                  sem_a, sem_b):
        wid = lax.axis_index("s") * NC + lax.axis_index("c")
        base = wid * pw

        def chunk(c, carry):
            off = base + c * C
            pltpu.sync_copy(idx_hbm.at[pl.ds(off, C)], idx_v)
            cp_a = pltpu.async_copy(umem_hbm.at[idx_v], rows_a, sem_a)
            cp_b = pltpu.async_copy(uout_hbm.at[idx_v], rows_b, sem_b)
            cp_a.wait()
            pltpu.sync_copy(rows_a, bmem_hbm.at[pl.ds(off, C)])
            cp_b.wait()
            pltpu.sync_copy(rows_b, bout_hbm.at[pl.ds(off, C)])
            return carry

        lax.fori_loop(0, nchunk, chunk, 0)

        ibase = wid * piw
        pltpu.sync_copy(uidx_hbm.at[pl.ds(ibase, piw)], uidx_v)
        pltpu.sync_copy(iidx_hbm.at[pl.ds(ibase, piw)], iidx_v)
        cp_a = pltpu.async_copy(umem_hbm.at[uidx_v], crows_a, sem_a)
        cp_b = pltpu.async_copy(imem_hbm.at[iidx_v], crows_b, sem_b)
        cp_a.wait()
        pltpu.sync_copy(crows_a, su_hbm.at[pl.ds(ibase, piw)])
        cp_b.wait()
        pltpu.sync_copy(crows_b, si_hbm.at[pl.ds(ibase, piw)])

    return sc_gather


def _dense_body(nb_ref, ub_ref, ib_ref, bmem_ref, bout_ref, su_ref, si_ref,
                whop_ref, bhop_ref, wout_ref, bout2_ref, w1_ref, score_ref):
    d = whop_ref.shape[0]

    def half(packed, idx):
        sel = (idx & 1) == 1
        return jnp.where(sel, packed[..., d:], packed[..., :d])

    mem = half(bmem_ref[...], nb_ref[...])
    nout = half(bout_ref[...], nb_ref[...])
    cu = half(su_ref[...], ub_ref[...])
    ci = half(si_ref[...], ib_ref[...])
    q = cu + ci

    def attend(query):
        s = jnp.sum(query[:, None, :] * mem, axis=2)
        s = s - jnp.max(s, axis=-1, keepdims=True)
        e = jnp.exp(s)
        a = e / jnp.sum(e, axis=-1, keepdims=True)
        return jnp.sum(a[:, :, None] * nout, axis=1)

    w0 = attend(q)
    q1 = jnp.dot(q, whop_ref[...], preferred_element_type=jnp.float32)
    q1 = jnp.maximum(q1 + bhop_ref[...] + w0, 0.0)
    w1 = attend(q1)

    aa = cu * ci
    z = (jnp.dot(aa, wout_ref[:d], preferred_element_type=jnp.float32)
         + jnp.dot(w1, wout_ref[d:], preferred_element_type=jnp.float32)
         + bout2_ref[...])
    score = jnp.maximum(jnp.sum(z * w1_ref[...], axis=1, keepdims=True), 0.0)
    score_ref[...] = score


def _dense(nb, ub, ib, bmem, bout, su, si, W_hop, b_hop, W_out, b_out, W_1,
           interpret=False):
    B = su.shape[0]
    L = bmem.shape[1]
    D = W_hop.shape[0]
    BB = 128
    grid = (B // BB,)
    return pl.pallas_call(
        _dense_body,
        grid=grid,
        in_specs=[
            pl.BlockSpec((BB, L, 1), lambda i: (i, 0, 0)),
            pl.BlockSpec((BB, 1), lambda i: (i, 0)),
            pl.BlockSpec((BB, 1), lambda i: (i, 0)),
            pl.BlockSpec((BB, L, 2 * D), lambda i: (i, 0, 0)),
            pl.BlockSpec((BB, L, 2 * D), lambda i: (i, 0, 0)),
            pl.BlockSpec((BB, 2 * D), lambda i: (i, 0)),
            pl.BlockSpec((BB, 2 * D), lambda i: (i, 0)),
            pl.BlockSpec((D, D), lambda i: (0, 0)),
            pl.BlockSpec((1, D), lambda i: (0, 0)),
            pl.BlockSpec((2 * D, D), lambda i: (0, 0)),
            pl.BlockSpec((1, D), lambda i: (0, 0)),
            pl.BlockSpec((1, D), lambda i: (0, 0)),
        ],
        out_specs=pl.BlockSpec((BB, 1), lambda i: (i, 0)),
        out_shape=jax.ShapeDtypeStruct((B, 1), jnp.float32),
        interpret=interpret,
    )(nb, ub, ib, bmem, bout, su, si, W_hop, b_hop, W_out, b_out, W_1)


def kernel(input_users, input_items, input_items_negative, input_neighborhoods,
           input_neighborhood_lengths, input_neighborhoods_negative,
           input_neighborhood_lengths_negative, user_memory, user_output,
           item_memory, W_hop, b_hop, W_out, b_out, W_1):
    B, L = input_neighborhoods.shape
    V, D = user_memory.shape

    nb = input_neighborhoods.astype(jnp.int32)
    ub = input_users.astype(jnp.int32).reshape(B, 1)
    ib = input_items.astype(jnp.int32).reshape(B, 1)

    sc_transpose = _make_sc_transpose(V, D)
    rem = V % 128
    nr = rem // 2
    umem2, uout2, imem2 = sc_transpose(
        user_memory.T, user_output.T, item_memory.T,
        user_memory[V - rem:].reshape(nr, 2 * D),
        user_output[V - rem:].reshape(nr, 2 * D),
        item_memory[V - rem:].reshape(nr, 2 * D))

    sc_gather = _make_sc_gather(B * L, B, 2 * D)
    bmem, bout, su, si = sc_gather(
        jnp.right_shift(nb, 1).reshape(-1), umem2, uout2,
        jnp.right_shift(ub, 1).reshape(-1), jnp.right_shift(ib, 1).reshape(-1),
        imem2)

    return _dense(nb.reshape(B, L, 1), ub, ib,
                  bmem.reshape(B, L, 2 * D), bout.reshape(B, L, 2 * D),
                  su, si, W_hop,
                  b_hop.reshape(1, D), W_out, b_out.reshape(1, D),
                  W_1.reshape(1, D))


# speedup vs baseline: 3.0566x; 2.2300x over previous
import functools

import jax
import jax.numpy as jnp
from jax import lax
from jax.experimental import pallas as pl
from jax.experimental.pallas import tpu as pltpu
from jax.experimental.pallas import tpu_sc as plsc


def _tc_transpose_body(a_ref, b_ref, c_ref, oa, ob, oc):
    for x, o in ((a_ref, oa), (b_ref, ob), (c_ref, oc)):
        t = jnp.transpose(x[...])
        w2 = t.shape[0] // 2
        o[...] = jnp.concatenate([t[:w2], t[w2:]], axis=1)


def _tc_transpose(umem_t, uout_t, imem_t, interpret=False):
    D, V = umem_t.shape
    W = 512
    nblk = pl.cdiv(V, W)
    in_spec = pl.BlockSpec((D, W), lambda i: (0, i))
    out_spec = pl.BlockSpec((W // 2, 2 * D), lambda i: (i, 0))
    sds = jax.ShapeDtypeStruct((nblk * (W // 2), 2 * D), jnp.float32)
    return pl.pallas_call(
        _tc_transpose_body,
        grid=(nblk,),
        in_specs=[in_spec] * 3,
        out_specs=[out_spec] * 3,
        out_shape=[sds] * 3,
        interpret=interpret,
    )(umem_t, uout_t, imem_t)


def _make_sc_gather(n_main, n_cur, D2):
    info = plsc.get_sparse_core_info()
    NC, NS = info.num_cores, info.num_subcores
    NW = NC * NS
    assert n_main % NW == 0 and n_cur % NW == 0
    pw = n_main // NW
    C = 256
    assert pw % C == 0
    nchunk = pw // C
    piw = n_cur // NW
    assert piw % 8 == 0

    mesh = plsc.VectorSubcoreMesh(core_axis_name="c", subcore_axis_name="s")

    @functools.partial(
        pl.kernel,
        mesh=mesh,
        compiler_params=pltpu.CompilerParams(use_tc_tiling_on_sc=True),
        out_type=[
            jax.ShapeDtypeStruct((n_main, D2), jnp.float32),
            jax.ShapeDtypeStruct((n_main, D2), jnp.float32),
            jax.ShapeDtypeStruct((n_cur, D2), jnp.float32),
            jax.ShapeDtypeStruct((n_cur, D2), jnp.float32),
        ],
        scratch_types=[
            pltpu.VMEM((C,), jnp.int32),
            pltpu.VMEM((C, D2), jnp.float32),
            pltpu.VMEM((C, D2), jnp.float32),
            pltpu.VMEM((piw,), jnp.int32),
            pltpu.VMEM((piw,), jnp.int32),
            pltpu.VMEM((piw, D2), jnp.float32),
            pltpu.VMEM((piw, D2), jnp.float32),
            pltpu.SemaphoreType.DMA,
            pltpu.SemaphoreType.DMA,
        ],
    )
    def sc_gather(idx_hbm, umem_hbm, uout_hbm, uidx_hbm, iidx_hbm, imem_hbm,
                  bmem_hbm, bout_hbm, su_hbm, si_hbm,
                  idx_v, rows_a, rows_b, uidx_v, iidx_v, crows_a, crows_b,
                  sem_a, sem_b):
        wid = lax.axis_index("s") * NC + lax.axis_index("c")
        base = wid * pw

        def chunk(c, carry):
            off = base + c * C
            pltpu.sync_copy(idx_hbm.at[pl.ds(off, C)], idx_v)
            cp_a = pltpu.async_copy(umem_hbm.at[idx_v], rows_a, sem_a)
            cp_b = pltpu.async_copy(uout_hbm.at[idx_v], rows_b, sem_b)
            cp_a.wait()
            pltpu.sync_copy(rows_a, bmem_hbm.at[pl.ds(off, C)])
            cp_b.wait()
            pltpu.sync_copy(rows_b, bout_hbm.at[pl.ds(off, C)])
            return carry

        lax.fori_loop(0, nchunk, chunk, 0)

        ibase = wid * piw
        pltpu.sync_copy(uidx_hbm.at[pl.ds(ibase, piw)], uidx_v)
        pltpu.sync_copy(iidx_hbm.at[pl.ds(ibase, piw)], iidx_v)
        cp_a = pltpu.async_copy(umem_hbm.at[uidx_v], crows_a, sem_a)
        cp_b = pltpu.async_copy(imem_hbm.at[iidx_v], crows_b, sem_b)
        cp_a.wait()
        pltpu.sync_copy(crows_a, su_hbm.at[pl.ds(ibase, piw)])
        cp_b.wait()
        pltpu.sync_copy(crows_b, si_hbm.at[pl.ds(ibase, piw)])

    return sc_gather


def _dense_body(nb_ref, ub_ref, ib_ref, bmem_ref, bout_ref, su_ref, si_ref,
                whop_ref, bhop_ref, wout_ref, bout2_ref, w1_ref, score_ref):
    d = whop_ref.shape[0]

    def half(packed, idx):
        sel = idx == 1
        return jnp.where(sel, packed[..., d:], packed[..., :d])

    mem = half(bmem_ref[...], nb_ref[...])
    nout = half(bout_ref[...], nb_ref[...])
    cu = half(su_ref[...], ub_ref[...])
    ci = half(si_ref[...], ib_ref[...])
    q = cu + ci

    def attend(query):
        s = jnp.sum(query[:, None, :] * mem, axis=2)
        s = s - jnp.max(s, axis=-1, keepdims=True)
        e = jnp.exp(s)
        a = e / jnp.sum(e, axis=-1, keepdims=True)
        return jnp.sum(a[:, :, None] * nout, axis=1)

    w0 = attend(q)
    q1 = jnp.dot(q, whop_ref[...], preferred_element_type=jnp.float32)
    q1 = jnp.maximum(q1 + bhop_ref[...] + w0, 0.0)
    w1 = attend(q1)

    aa = cu * ci
    z = (jnp.dot(aa, wout_ref[:d], preferred_element_type=jnp.float32)
         + jnp.dot(w1, wout_ref[d:], preferred_element_type=jnp.float32)
         + bout2_ref[...])
    score = jnp.maximum(jnp.sum(z * w1_ref[...], axis=1, keepdims=True), 0.0)
    score_ref[...] = score


def _dense(nb, ub, ib, bmem, bout, su, si, W_hop, b_hop, W_out, b_out, W_1,
           interpret=False):
    B = su.shape[0]
    L = bmem.shape[1]
    D = W_hop.shape[0]
    BB = 128
    grid = (B // BB,)
    return pl.pallas_call(
        _dense_body,
        grid=grid,
        in_specs=[
            pl.BlockSpec((BB, L, 1), lambda i: (i, 0, 0)),
            pl.BlockSpec((BB, 1), lambda i: (i, 0)),
            pl.BlockSpec((BB, 1), lambda i: (i, 0)),
            pl.BlockSpec((BB, L, 2 * D), lambda i: (i, 0, 0)),
            pl.BlockSpec((BB, L, 2 * D), lambda i: (i, 0, 0)),
            pl.BlockSpec((BB, 2 * D), lambda i: (i, 0)),
            pl.BlockSpec((BB, 2 * D), lambda i: (i, 0)),
            pl.BlockSpec((D, D), lambda i: (0, 0)),
            pl.BlockSpec((1, D), lambda i: (0, 0)),
            pl.BlockSpec((2 * D, D), lambda i: (0, 0)),
            pl.BlockSpec((1, D), lambda i: (0, 0)),
            pl.BlockSpec((1, D), lambda i: (0, 0)),
        ],
        out_specs=pl.BlockSpec((BB, 1), lambda i: (i, 0)),
        out_shape=jax.ShapeDtypeStruct((B, 1), jnp.float32),
        interpret=interpret,
    )(nb, ub, ib, bmem, bout, su, si, W_hop, b_hop, W_out, b_out, W_1)


def kernel(input_users, input_items, input_items_negative, input_neighborhoods,
           input_neighborhood_lengths, input_neighborhoods_negative,
           input_neighborhood_lengths_negative, user_memory, user_output,
           item_memory, W_hop, b_hop, W_out, b_out, W_1):
    B, L = input_neighborhoods.shape
    V, D = user_memory.shape

    nb = input_neighborhoods.astype(jnp.int32)
    ub = input_users.astype(jnp.int32).reshape(B, 1)
    ib = input_items.astype(jnp.int32).reshape(B, 1)
    def pk(x):
        return jnp.left_shift(jnp.right_shift(x, 9), 8) | (x & 255)

    def sel(x):
        return jnp.right_shift(x, 8) & 1

    nb_sel, ub_sel, ib_sel = sel(nb), sel(ub), sel(ib)

    umem2, uout2, imem2 = _tc_transpose(
        user_memory.T, user_output.T, item_memory.T)

    sc_gather = _make_sc_gather(B * L, B, 2 * D)
    bmem, bout, su, si = sc_gather(
        pk(nb).reshape(-1), umem2, uout2,
        pk(ub).reshape(-1), pk(ib).reshape(-1),
        imem2)

    return _dense(nb_sel.reshape(B, L, 1), ub_sel, ib_sel,
                  bmem.reshape(B, L, 2 * D), bout.reshape(B, L, 2 * D),
                  su, si, W_hop,
                  b_hop.reshape(1, D), W_out, b_out.reshape(1, D),
                  W_1.reshape(1, D))


# speedup vs baseline: 4.4657x; 1.4610x over previous
import functools

import jax
import jax.numpy as jnp
from jax import lax
from jax.experimental import pallas as pl
from jax.experimental.pallas import tpu as pltpu
from jax.experimental.pallas import tpu_sc as plsc


def _tc_transpose_body(a_ref, b_ref, c_ref, oa, ob, oc):
    for x, o in ((a_ref, oa), (b_ref, ob), (c_ref, oc)):
        t = jnp.transpose(x[...])
        w2 = t.shape[0] // 2
        o[...] = jnp.concatenate([t[:w2], t[w2:]], axis=1)


_TP_W = 2048


def _tc_transpose(umem_t, uout_t, imem_t, interpret=False):
    D, V = umem_t.shape
    W = _TP_W
    nblk = pl.cdiv(V, W)
    in_spec = pl.BlockSpec((D, W), lambda i: (0, i))
    out_spec = pl.BlockSpec((W // 2, 2 * D), lambda i: (i, 0))
    sds = jax.ShapeDtypeStruct((nblk * (W // 2), 2 * D), jnp.float32)
    return pl.pallas_call(
        _tc_transpose_body,
        grid=(nblk,),
        in_specs=[in_spec] * 3,
        out_specs=[out_spec] * 3,
        out_shape=[sds] * 3,
        interpret=interpret,
    )(umem_t, uout_t, imem_t)


def _make_sc_gather(n_main, n_cur, D2):
    info = plsc.get_sparse_core_info()
    NC, NS = info.num_cores, info.num_subcores
    NW = NC * NS
    assert n_main % NW == 0 and n_cur % NW == 0
    pw = n_main // NW
    C = 256
    assert pw % C == 0
    nchunk = pw // C
    piw = n_cur // NW
    assert piw % 8 == 0

    mesh = plsc.VectorSubcoreMesh(core_axis_name="c", subcore_axis_name="s")

    @functools.partial(
        pl.kernel,
        mesh=mesh,
        compiler_params=pltpu.CompilerParams(use_tc_tiling_on_sc=True),
        out_type=[
            jax.ShapeDtypeStruct((n_main, D2), jnp.float32),
            jax.ShapeDtypeStruct((n_main, D2), jnp.float32),
            jax.ShapeDtypeStruct((n_cur, D2), jnp.float32),
            jax.ShapeDtypeStruct((n_cur, D2), jnp.float32),
        ],
        scratch_types=[
            pltpu.VMEM((C,), jnp.int32),
            pltpu.VMEM((C, D2), jnp.float32),
            pltpu.VMEM((C, D2), jnp.float32),
            pltpu.VMEM((piw,), jnp.int32),
            pltpu.VMEM((piw,), jnp.int32),
            pltpu.VMEM((piw, D2), jnp.float32),
            pltpu.VMEM((piw, D2), jnp.float32),
            pltpu.SemaphoreType.DMA,
            pltpu.SemaphoreType.DMA,
        ],
    )
    def sc_gather(idx_hbm, umem_hbm, uout_hbm, uidx_hbm, iidx_hbm, imem_hbm,
                  bmem_hbm, bout_hbm, su_hbm, si_hbm,
                  idx_v, rows_a, rows_b, uidx_v, iidx_v, crows_a, crows_b,
                  sem_a, sem_b):
        wid = lax.axis_index("s") * NC + lax.axis_index("c")
        base = wid * pw

        def chunk(c, carry):
            off = base + c * C
            pltpu.sync_copy(idx_hbm.at[pl.ds(off, C)], idx_v)
            cp_a = pltpu.async_copy(umem_hbm.at[idx_v], rows_a, sem_a)
            cp_b = pltpu.async_copy(uout_hbm.at[idx_v], rows_b, sem_b)
            cp_a.wait()
            pltpu.sync_copy(rows_a, bmem_hbm.at[pl.ds(off, C)])
            cp_b.wait()
            pltpu.sync_copy(rows_b, bout_hbm.at[pl.ds(off, C)])
            return carry

        lax.fori_loop(0, nchunk, chunk, 0)

        ibase = wid * piw
        pltpu.sync_copy(uidx_hbm.at[pl.ds(ibase, piw)], uidx_v)
        pltpu.sync_copy(iidx_hbm.at[pl.ds(ibase, piw)], iidx_v)
        cp_a = pltpu.async_copy(umem_hbm.at[uidx_v], crows_a, sem_a)
        cp_b = pltpu.async_copy(imem_hbm.at[iidx_v], crows_b, sem_b)
        cp_a.wait()
        pltpu.sync_copy(crows_a, su_hbm.at[pl.ds(ibase, piw)])
        cp_b.wait()
        pltpu.sync_copy(crows_b, si_hbm.at[pl.ds(ibase, piw)])

    return sc_gather


def _dense_body(nb_ref, ub_ref, ib_ref, bmem_ref, bout_ref, su_ref, si_ref,
                whop_ref, bhop_ref, wout_ref, bout2_ref, w1_ref, score_ref):
    d = whop_ref.shape[0]

    def half(packed, idx):
        sel = idx == 1
        return jnp.where(sel, packed[..., d:], packed[..., :d])

    mem = half(bmem_ref[...], nb_ref[...])
    nout = half(bout_ref[...], nb_ref[...])
    cu = half(su_ref[...], ub_ref[...])
    ci = half(si_ref[...], ib_ref[...])
    q = cu + ci

    def attend(query):
        s = jnp.sum(query[:, None, :] * mem, axis=2)
        s = s - jnp.max(s, axis=-1, keepdims=True)
        e = jnp.exp(s)
        a = e / jnp.sum(e, axis=-1, keepdims=True)
        return jnp.sum(a[:, :, None] * nout, axis=1)

    w0 = attend(q)
    q1 = jnp.dot(q, whop_ref[...], preferred_element_type=jnp.float32)
    q1 = jnp.maximum(q1 + bhop_ref[...] + w0, 0.0)
    w1 = attend(q1)

    aa = cu * ci
    z = (jnp.dot(aa, wout_ref[:d], preferred_element_type=jnp.float32)
         + jnp.dot(w1, wout_ref[d:], preferred_element_type=jnp.float32)
         + bout2_ref[...])
    score = jnp.maximum(jnp.sum(z * w1_ref[...], axis=1, keepdims=True), 0.0)
    score_ref[...] = score


def _dense(nb, ub, ib, bmem, bout, su, si, W_hop, b_hop, W_out, b_out, W_1,
           interpret=False):
    B = su.shape[0]
    L = bmem.shape[1]
    D = W_hop.shape[0]
    BB = 128
    grid = (B // BB,)
    return pl.pallas_call(
        _dense_body,
        grid=grid,
        in_specs=[
            pl.BlockSpec((BB, L, 1), lambda i: (i, 0, 0)),
            pl.BlockSpec((BB, 1), lambda i: (i, 0)),
            pl.BlockSpec((BB, 1), lambda i: (i, 0)),
            pl.BlockSpec((BB, L, 2 * D), lambda i: (i, 0, 0)),
            pl.BlockSpec((BB, L, 2 * D), lambda i: (i, 0, 0)),
            pl.BlockSpec((BB, 2 * D), lambda i: (i, 0)),
            pl.BlockSpec((BB, 2 * D), lambda i: (i, 0)),
            pl.BlockSpec((D, D), lambda i: (0, 0)),
            pl.BlockSpec((1, D), lambda i: (0, 0)),
            pl.BlockSpec((2 * D, D), lambda i: (0, 0)),
            pl.BlockSpec((1, D), lambda i: (0, 0)),
            pl.BlockSpec((1, D), lambda i: (0, 0)),
        ],
        out_specs=pl.BlockSpec((BB, 1), lambda i: (i, 0)),
        out_shape=jax.ShapeDtypeStruct((B, 1), jnp.float32),
        interpret=interpret,
    )(nb, ub, ib, bmem, bout, su, si, W_hop, b_hop, W_out, b_out, W_1)


def kernel(input_users, input_items, input_items_negative, input_neighborhoods,
           input_neighborhood_lengths, input_neighborhoods_negative,
           input_neighborhood_lengths_negative, user_memory, user_output,
           item_memory, W_hop, b_hop, W_out, b_out, W_1):
    B, L = input_neighborhoods.shape
    V, D = user_memory.shape

    nb = input_neighborhoods.astype(jnp.int32)
    ub = input_users.astype(jnp.int32).reshape(B, 1)
    ib = input_items.astype(jnp.int32).reshape(B, 1)
    hw = _TP_W // 2

    def pk(x):
        return (x // _TP_W) * hw + (x % hw)

    def sel(x):
        return (x // hw) & 1

    nb_sel, ub_sel, ib_sel = sel(nb), sel(ub), sel(ib)

    umem2, uout2, imem2 = _tc_transpose(
        user_memory.T, user_output.T, item_memory.T)

    sc_gather = _make_sc_gather(B * L, B, 2 * D)
    bmem, bout, su, si = sc_gather(
        pk(nb).reshape(-1), umem2, uout2,
        pk(ub).reshape(-1), pk(ib).reshape(-1),
        imem2)

    return _dense(nb_sel.reshape(B, L, 1), ub_sel, ib_sel,
                  bmem.reshape(B, L, 2 * D), bout.reshape(B, L, 2 * D),
                  su, si, W_hop,
                  b_hop.reshape(1, D), W_out, b_out.reshape(1, D),
                  W_1.reshape(1, D))


# speedup vs baseline: 5.0485x; 1.1305x over previous
import functools

import jax
import jax.numpy as jnp
from jax import lax
from jax.experimental import pallas as pl
from jax.experimental.pallas import tpu as pltpu
from jax.experimental.pallas import tpu_sc as plsc


def _tc_transpose_body(a_ref, b_ref, c_ref, oa, ob, oc):
    for x, o in ((a_ref, oa), (b_ref, ob), (c_ref, oc)):
        t = jnp.transpose(x[...])
        w2 = t.shape[0] // 2
        o[...] = jnp.concatenate([t[:w2], t[w2:]], axis=1)


_TP_W = 8192


def _tc_transpose(umem_t, uout_t, imem_t, interpret=False):
    D, V = umem_t.shape
    W = _TP_W
    nblk = pl.cdiv(V, W)
    in_spec = pl.BlockSpec((D, W), lambda i: (0, i))
    out_spec = pl.BlockSpec((W // 2, 2 * D), lambda i: (i, 0))
    sds = jax.ShapeDtypeStruct((nblk * (W // 2), 2 * D), jnp.float32)
    return pl.pallas_call(
        _tc_transpose_body,
        grid=(nblk,),
        in_specs=[in_spec] * 3,
        out_specs=[out_spec] * 3,
        out_shape=[sds] * 3,
        interpret=interpret,
    )(umem_t, uout_t, imem_t)


def _make_sc_gather(n_main, n_cur, D2):
    info = plsc.get_sparse_core_info()
    NC, NS = info.num_cores, info.num_subcores
    NW = NC * NS
    assert n_main % NW == 0 and n_cur % NW == 0
    pw = n_main // NW
    C = 256
    assert pw % C == 0
    nchunk = pw // C
    piw = n_cur // NW
    assert piw % 8 == 0

    mesh = plsc.VectorSubcoreMesh(core_axis_name="c", subcore_axis_name="s")

    @functools.partial(
        pl.kernel,
        mesh=mesh,
        compiler_params=pltpu.CompilerParams(use_tc_tiling_on_sc=True),
        out_type=[
            jax.ShapeDtypeStruct((n_main, D2), jnp.float32),
            jax.ShapeDtypeStruct((n_main, D2), jnp.float32),
            jax.ShapeDtypeStruct((n_cur, D2), jnp.float32),
            jax.ShapeDtypeStruct((n_cur, D2), jnp.float32),
        ],
        scratch_types=[
            pltpu.VMEM((C,), jnp.int32),
            pltpu.VMEM((C, D2), jnp.float32),
            pltpu.VMEM((C, D2), jnp.float32),
            pltpu.VMEM((piw,), jnp.int32),
            pltpu.VMEM((piw,), jnp.int32),
            pltpu.VMEM((piw, D2), jnp.float32),
            pltpu.VMEM((piw, D2), jnp.float32),
            pltpu.SemaphoreType.DMA,
            pltpu.SemaphoreType.DMA,
        ],
    )
    def sc_gather(idx_hbm, umem_hbm, uout_hbm, uidx_hbm, iidx_hbm, imem_hbm,
                  bmem_hbm, bout_hbm, su_hbm, si_hbm,
                  idx_v, rows_a, rows_b, uidx_v, iidx_v, crows_a, crows_b,
                  sem_a, sem_b):
        wid = lax.axis_index("s") * NC + lax.axis_index("c")
        base = wid * pw

        def chunk(c, carry):
            off = base + c * C
            pltpu.sync_copy(idx_hbm.at[pl.ds(off, C)], idx_v)
            cp_a = pltpu.async_copy(umem_hbm.at[idx_v], rows_a, sem_a)
            cp_b = pltpu.async_copy(uout_hbm.at[idx_v], rows_b, sem_b)
            cp_a.wait()
            pltpu.sync_copy(rows_a, bmem_hbm.at[pl.ds(off, C)])
            cp_b.wait()
            pltpu.sync_copy(rows_b, bout_hbm.at[pl.ds(off, C)])
            return carry

        lax.fori_loop(0, nchunk, chunk, 0)

        ibase = wid * piw
        pltpu.sync_copy(uidx_hbm.at[pl.ds(ibase, piw)], uidx_v)
        pltpu.sync_copy(iidx_hbm.at[pl.ds(ibase, piw)], iidx_v)
        cp_a = pltpu.async_copy(umem_hbm.at[uidx_v], crows_a, sem_a)
        cp_b = pltpu.async_copy(imem_hbm.at[iidx_v], crows_b, sem_b)
        cp_a.wait()
        pltpu.sync_copy(crows_a, su_hbm.at[pl.ds(ibase, piw)])
        cp_b.wait()
        pltpu.sync_copy(crows_b, si_hbm.at[pl.ds(ibase, piw)])

    return sc_gather


def _dense_body(nb_ref, ub_ref, ib_ref, bmem_ref, bout_ref, su_ref, si_ref,
                whop_ref, bhop_ref, wout_ref, bout2_ref, w1_ref, score_ref):
    d = whop_ref.shape[0]

    def half(packed, idx):
        sel = idx == 1
        return jnp.where(sel, packed[..., d:], packed[..., :d])

    mem = half(bmem_ref[...], nb_ref[...])
    nout = half(bout_ref[...], nb_ref[...])
    cu = half(su_ref[...], ub_ref[...])
    ci = half(si_ref[...], ib_ref[...])
    q = cu + ci

    def attend(query):
        s = jnp.sum(query[:, None, :] * mem, axis=2)
        s = s - jnp.max(s, axis=-1, keepdims=True)
        e = jnp.exp(s)
        a = e / jnp.sum(e, axis=-1, keepdims=True)
        return jnp.sum(a[:, :, None] * nout, axis=1)

    w0 = attend(q)
    q1 = jnp.dot(q, whop_ref[...], preferred_element_type=jnp.float32)
    q1 = jnp.maximum(q1 + bhop_ref[...] + w0, 0.0)
    w1 = attend(q1)

    aa = cu * ci
    z = (jnp.dot(aa, wout_ref[:d], preferred_element_type=jnp.float32)
         + jnp.dot(w1, wout_ref[d:], preferred_element_type=jnp.float32)
         + bout2_ref[...])
    score = jnp.maximum(jnp.sum(z * w1_ref[...], axis=1, keepdims=True), 0.0)
    score_ref[...] = score


def _dense(nb, ub, ib, bmem, bout, su, si, W_hop, b_hop, W_out, b_out, W_1,
           interpret=False):
    B = su.shape[0]
    L = bmem.shape[1]
    D = W_hop.shape[0]
    BB = 128
    grid = (B // BB,)
    return pl.pallas_call(
        _dense_body,
        grid=grid,
        in_specs=[
            pl.BlockSpec((BB, L, 1), lambda i: (i, 0, 0)),
            pl.BlockSpec((BB, 1), lambda i: (i, 0)),
            pl.BlockSpec((BB, 1), lambda i: (i, 0)),
            pl.BlockSpec((BB, L, 2 * D), lambda i: (i, 0, 0)),
            pl.BlockSpec((BB, L, 2 * D), lambda i: (i, 0, 0)),
            pl.BlockSpec((BB, 2 * D), lambda i: (i, 0)),
            pl.BlockSpec((BB, 2 * D), lambda i: (i, 0)),
            pl.BlockSpec((D, D), lambda i: (0, 0)),
            pl.BlockSpec((1, D), lambda i: (0, 0)),
            pl.BlockSpec((2 * D, D), lambda i: (0, 0)),
            pl.BlockSpec((1, D), lambda i: (0, 0)),
            pl.BlockSpec((1, D), lambda i: (0, 0)),
        ],
        out_specs=pl.BlockSpec((BB, 1), lambda i: (i, 0)),
        out_shape=jax.ShapeDtypeStruct((B, 1), jnp.float32),
        interpret=interpret,
    )(nb, ub, ib, bmem, bout, su, si, W_hop, b_hop, W_out, b_out, W_1)


def kernel(input_users, input_items, input_items_negative, input_neighborhoods,
           input_neighborhood_lengths, input_neighborhoods_negative,
           input_neighborhood_lengths_negative, user_memory, user_output,
           item_memory, W_hop, b_hop, W_out, b_out, W_1):
    B, L = input_neighborhoods.shape
    V, D = user_memory.shape

    nb = input_neighborhoods.astype(jnp.int32)
    ub = input_users.astype(jnp.int32).reshape(B, 1)
    ib = input_items.astype(jnp.int32).reshape(B, 1)
    hw = _TP_W // 2

    def pk(x):
        return (x // _TP_W) * hw + (x % hw)

    def sel(x):
        return (x // hw) & 1

    nb_sel, ub_sel, ib_sel = sel(nb), sel(ub), sel(ib)

    umem2, uout2, imem2 = _tc_transpose(
        user_memory.T, user_output.T, item_memory.T)

    sc_gather = _make_sc_gather(B * L, B, 2 * D)
    bmem, bout, su, si = sc_gather(
        pk(nb).reshape(-1), umem2, uout2,
        pk(ub).reshape(-1), pk(ib).reshape(-1),
        imem2)

    return _dense(nb_sel.reshape(B, L, 1), ub_sel, ib_sel,
                  bmem.reshape(B, L, 2 * D), bout.reshape(B, L, 2 * D),
                  su, si, W_hop,
                  b_hop.reshape(1, D), W_out, b_out.reshape(1, D),
                  W_1.reshape(1, D))


# speedup vs baseline: 5.0679x; 1.0038x over previous
import functools

import jax
import jax.numpy as jnp
from jax import lax
from jax.experimental import pallas as pl
from jax.experimental.pallas import tpu as pltpu
from jax.experimental.pallas import tpu_sc as plsc


def _tc_transpose_body(a_ref, b_ref, c_ref, oa, ob, oc):
    for x, o in ((a_ref, oa), (b_ref, ob), (c_ref, oc)):
        t = jnp.transpose(x[...])
        w2 = t.shape[0] // 2
        o[...] = jnp.concatenate([t[:w2], t[w2:]], axis=1)


_TP_W = 12288


def _tc_transpose(umem_t, uout_t, imem_t, interpret=False):
    D, V = umem_t.shape
    W = _TP_W
    nblk = pl.cdiv(V, W)
    in_spec = pl.BlockSpec((D, W), lambda i: (0, i))
    out_spec = pl.BlockSpec((W // 2, 2 * D), lambda i: (i, 0))
    sds = jax.ShapeDtypeStruct((nblk * (W // 2), 2 * D), jnp.float32)
    return pl.pallas_call(
        _tc_transpose_body,
        grid=(nblk,),
        in_specs=[in_spec] * 3,
        out_specs=[out_spec] * 3,
        out_shape=[sds] * 3,
        interpret=interpret,
    )(umem_t, uout_t, imem_t)


def _make_sc_gather(n_main, n_cur, D2):
    info = plsc.get_sparse_core_info()
    NC, NS = info.num_cores, info.num_subcores
    NW = NC * NS
    assert n_main % NW == 0 and n_cur % NW == 0
    pw = n_main // NW
    C = 256
    assert pw % C == 0
    nchunk = pw // C
    piw = n_cur // NW
    assert piw % 8 == 0

    mesh = plsc.VectorSubcoreMesh(core_axis_name="c", subcore_axis_name="s")

    @functools.partial(
        pl.kernel,
        mesh=mesh,
        compiler_params=pltpu.CompilerParams(use_tc_tiling_on_sc=True),
        out_type=[
            jax.ShapeDtypeStruct((n_main, D2), jnp.float32),
            jax.ShapeDtypeStruct((n_main, D2), jnp.float32),
            jax.ShapeDtypeStruct((n_cur, D2), jnp.float32),
            jax.ShapeDtypeStruct((n_cur, D2), jnp.float32),
        ],
        scratch_types=[
            pltpu.VMEM((C,), jnp.int32),
            pltpu.VMEM((C, D2), jnp.float32),
            pltpu.VMEM((C, D2), jnp.float32),
            pltpu.VMEM((piw,), jnp.int32),
            pltpu.VMEM((piw,), jnp.int32),
            pltpu.VMEM((piw, D2), jnp.float32),
            pltpu.VMEM((piw, D2), jnp.float32),
            pltpu.SemaphoreType.DMA,
            pltpu.SemaphoreType.DMA,
        ],
    )
    def sc_gather(idx_hbm, umem_hbm, uout_hbm, uidx_hbm, iidx_hbm, imem_hbm,
                  bmem_hbm, bout_hbm, su_hbm, si_hbm,
                  idx_v, rows_a, rows_b, uidx_v, iidx_v, crows_a, crows_b,
                  sem_a, sem_b):
        wid = lax.axis_index("s") * NC + lax.axis_index("c")
        base = wid * pw

        def chunk(c, carry):
            off = base + c * C
            pltpu.sync_copy(idx_hbm.at[pl.ds(off, C)], idx_v)
            cp_a = pltpu.async_copy(umem_hbm.at[idx_v], rows_a, sem_a)
            cp_b = pltpu.async_copy(uout_hbm.at[idx_v], rows_b, sem_b)
            cp_a.wait()
            pltpu.sync_copy(rows_a, bmem_hbm.at[pl.ds(off, C)])
            cp_b.wait()
            pltpu.sync_copy(rows_b, bout_hbm.at[pl.ds(off, C)])
            return carry

        lax.fori_loop(0, nchunk, chunk, 0)

        ibase = wid * piw
        pltpu.sync_copy(uidx_hbm.at[pl.ds(ibase, piw)], uidx_v)
        pltpu.sync_copy(iidx_hbm.at[pl.ds(ibase, piw)], iidx_v)
        cp_a = pltpu.async_copy(umem_hbm.at[uidx_v], crows_a, sem_a)
        cp_b = pltpu.async_copy(imem_hbm.at[iidx_v], crows_b, sem_b)
        cp_a.wait()
        pltpu.sync_copy(crows_a, su_hbm.at[pl.ds(ibase, piw)])
        cp_b.wait()
        pltpu.sync_copy(crows_b, si_hbm.at[pl.ds(ibase, piw)])

    return sc_gather


def _dense_body(nb_ref, ub_ref, ib_ref, bmem_ref, bout_ref, su_ref, si_ref,
                whop_ref, bhop_ref, wout_ref, bout2_ref, w1_ref, score_ref):
    d = whop_ref.shape[0]

    def half(packed, idx):
        sel = idx == 1
        return jnp.where(sel, packed[..., d:], packed[..., :d])

    mem = half(bmem_ref[...], nb_ref[...])
    nout = half(bout_ref[...], nb_ref[...])
    cu = half(su_ref[...], ub_ref[...])
    ci = half(si_ref[...], ib_ref[...])
    q = cu + ci

    def attend(query):
        s = jnp.sum(query[:, None, :] * mem, axis=2)
        s = s - jnp.max(s, axis=-1, keepdims=True)
        e = jnp.exp(s)
        a = e / jnp.sum(e, axis=-1, keepdims=True)
        return jnp.sum(a[:, :, None] * nout, axis=1)

    w0 = attend(q)
    q1 = jnp.dot(q, whop_ref[...], preferred_element_type=jnp.float32)
    q1 = jnp.maximum(q1 + bhop_ref[...] + w0, 0.0)
    w1 = attend(q1)

    aa = cu * ci
    z = (jnp.dot(aa, wout_ref[:d], preferred_element_type=jnp.float32)
         + jnp.dot(w1, wout_ref[d:], preferred_element_type=jnp.float32)
         + bout2_ref[...])
    score = jnp.maximum(jnp.sum(z * w1_ref[...], axis=1, keepdims=True), 0.0)
    score_ref[...] = score


def _dense(nb, ub, ib, bmem, bout, su, si, W_hop, b_hop, W_out, b_out, W_1,
           interpret=False):
    B = su.shape[0]
    L = bmem.shape[1]
    D = W_hop.shape[0]
    BB = 128
    grid = (B // BB,)
    return pl.pallas_call(
        _dense_body,
        grid=grid,
        in_specs=[
            pl.BlockSpec((BB, L, 1), lambda i: (i, 0, 0)),
            pl.BlockSpec((BB, 1), lambda i: (i, 0)),
            pl.BlockSpec((BB, 1), lambda i: (i, 0)),
            pl.BlockSpec((BB, L, 2 * D), lambda i: (i, 0, 0)),
            pl.BlockSpec((BB, L, 2 * D), lambda i: (i, 0, 0)),
            pl.BlockSpec((BB, 2 * D), lambda i: (i, 0)),
            pl.BlockSpec((BB, 2 * D), lambda i: (i, 0)),
            pl.BlockSpec((D, D), lambda i: (0, 0)),
            pl.BlockSpec((1, D), lambda i: (0, 0)),
            pl.BlockSpec((2 * D, D), lambda i: (0, 0)),
            pl.BlockSpec((1, D), lambda i: (0, 0)),
            pl.BlockSpec((1, D), lambda i: (0, 0)),
        ],
        out_specs=pl.BlockSpec((BB, 1), lambda i: (i, 0)),
        out_shape=jax.ShapeDtypeStruct((B, 1), jnp.float32),
        interpret=interpret,
    )(nb, ub, ib, bmem, bout, su, si, W_hop, b_hop, W_out, b_out, W_1)


def kernel(input_users, input_items, input_items_negative, input_neighborhoods,
           input_neighborhood_lengths, input_neighborhoods_negative,
           input_neighborhood_lengths_negative, user_memory, user_output,
           item_memory, W_hop, b_hop, W_out, b_out, W_1):
    B, L = input_neighborhoods.shape
    V, D = user_memory.shape

    nb = input_neighborhoods.astype(jnp.int32)
    ub = input_users.astype(jnp.int32).reshape(B, 1)
    ib = input_items.astype(jnp.int32).reshape(B, 1)
    hw = _TP_W // 2

    def pk(x):
        return (x // _TP_W) * hw + (x % hw)

    def sel(x):
        return (x // hw) & 1

    nb_sel, ub_sel, ib_sel = sel(nb), sel(ub), sel(ib)

    umem2, uout2, imem2 = _tc_transpose(
        user_memory.T, user_output.T, item_memory.T)

    sc_gather = _make_sc_gather(B * L, B, 2 * D)
    bmem, bout, su, si = sc_gather(
        pk(nb).reshape(-1), umem2, uout2,
        pk(ub).reshape(-1), pk(ib).reshape(-1),
        imem2)

    return _dense(nb_sel.reshape(B, L, 1), ub_sel, ib_sel,
                  bmem.reshape(B, L, 2 * D), bout.reshape(B, L, 2 * D),
                  su, si, W_hop,
                  b_hop.reshape(1, D), W_out, b_out.reshape(1, D),
                  W_1.reshape(1, D))


# speedup vs baseline: 5.1109x; 1.0085x over previous
import functools

import jax
import jax.numpy as jnp
from jax import lax
from jax.experimental import pallas as pl
from jax.experimental.pallas import tpu as pltpu
from jax.experimental.pallas import tpu_sc as plsc


def _tc_transpose_body(a_ref, b_ref, c_ref, oa, ob, oc):
    for x, o in ((a_ref, oa), (b_ref, ob), (c_ref, oc)):
        t = jnp.transpose(x[...])
        w2 = t.shape[0] // 2
        o[...] = jnp.concatenate([t[:w2], t[w2:]], axis=1)


_TP_W = 12288


def _tc_transpose(umem_t, uout_t, imem_t, interpret=False):
    D, V = umem_t.shape
    W = _TP_W
    nblk = pl.cdiv(V, W)
    in_spec = pl.BlockSpec((D, W), lambda i: (0, i))
    out_spec = pl.BlockSpec((W // 2, 2 * D), lambda i: (i, 0))
    sds = jax.ShapeDtypeStruct((nblk * (W // 2), 2 * D), jnp.float32)
    return pl.pallas_call(
        _tc_transpose_body,
        grid=(nblk,),
        in_specs=[in_spec] * 3,
        out_specs=[out_spec] * 3,
        out_shape=[sds] * 3,
        interpret=interpret,
    )(umem_t, uout_t, imem_t)


def _make_sc_gather(n_main, n_cur, D2):
    info = plsc.get_sparse_core_info()
    NC, NS = info.num_cores, info.num_subcores
    NW = NC * NS
    assert n_main % NW == 0 and n_cur % NW == 0
    pw = n_main // NW
    C = 128
    assert pw % C == 0
    nchunk = pw // C
    piw = n_cur // NW
    assert piw % 8 == 0

    mesh = plsc.VectorSubcoreMesh(core_axis_name="c", subcore_axis_name="s")

    @functools.partial(
        pl.kernel,
        mesh=mesh,
        compiler_params=pltpu.CompilerParams(use_tc_tiling_on_sc=True),
        out_type=[
            jax.ShapeDtypeStruct((n_main, D2), jnp.float32),
            jax.ShapeDtypeStruct((n_main, D2), jnp.float32),
            jax.ShapeDtypeStruct((n_cur, D2), jnp.float32),
            jax.ShapeDtypeStruct((n_cur, D2), jnp.float32),
        ],
        scratch_types=[
            pltpu.VMEM((pw,), jnp.int32),
            pltpu.VMEM((C, D2), jnp.float32),
            pltpu.VMEM((C, D2), jnp.float32),
            pltpu.VMEM((C, D2), jnp.float32),
            pltpu.VMEM((C, D2), jnp.float32),
            pltpu.VMEM((piw,), jnp.int32),
            pltpu.VMEM((piw,), jnp.int32),
            pltpu.VMEM((piw, D2), jnp.float32),
            pltpu.VMEM((piw, D2), jnp.float32),
            pltpu.SemaphoreType.DMA,
            pltpu.SemaphoreType.DMA,
            pltpu.SemaphoreType.DMA,
            pltpu.SemaphoreType.DMA,
        ],
    )
    def sc_gather(idx_hbm, umem_hbm, uout_hbm, uidx_hbm, iidx_hbm, imem_hbm,
                  bmem_hbm, bout_hbm, su_hbm, si_hbm,
                  idx_all, rows_a0, rows_b0, rows_a1, rows_b1,
                  uidx_v, iidx_v, crows_a, crows_b,
                  sem_a, sem_b, sw0, sw1):
        wid = lax.axis_index("s") * NC + lax.axis_index("c")
        base = wid * pw
        bufs = ((rows_a0, rows_b0, sem_a, sw0), (rows_a1, rows_b1, sem_b, sw1))

        pltpu.sync_copy(idx_hbm.at[pl.ds(base, pw)], idx_all)

        def idxs(k):
            return idx_all.at[pl.ds(k * C, C)]

        def g_pair(k):
            ra, rb, sg, _ = bufs[k % 2]
            return ((umem_hbm.at[idxs(k)], ra, sg),
                    (uout_hbm.at[idxs(k)], rb, sg))

        def w_pair(k):
            ra, rb, _, sw = bufs[k % 2]
            off = base + k * C
            return ((ra, bmem_hbm.at[pl.ds(off, C)], sw),
                    (rb, bout_hbm.at[pl.ds(off, C)], sw))

        for src, dst, sem in g_pair(0):
            pltpu.async_copy(src, dst, sem)
        for k in range(nchunk):
            for src, dst, sem in g_pair(k):
                pltpu.make_async_copy(src, dst, sem).wait()
            for src, dst, sem in w_pair(k):
                pltpu.async_copy(src, dst, sem)
            if k + 1 < nchunk:
                if k >= 1:
                    for src, dst, sem in w_pair(k - 1):
                        pltpu.make_async_copy(src, dst, sem).wait()
                for src, dst, sem in g_pair(k + 1):
                    pltpu.async_copy(src, dst, sem)
        for k in (nchunk - 2, nchunk - 1):
            if k >= 0:
                for src, dst, sem in w_pair(k):
                    pltpu.make_async_copy(src, dst, sem).wait()

        ibase = wid * piw
        pltpu.sync_copy(uidx_hbm.at[pl.ds(ibase, piw)], uidx_v)
        pltpu.sync_copy(iidx_hbm.at[pl.ds(ibase, piw)], iidx_v)
        cp_a = pltpu.async_copy(umem_hbm.at[uidx_v], crows_a, sem_a)
        cp_b = pltpu.async_copy(imem_hbm.at[iidx_v], crows_b, sem_b)
        cp_a.wait()
        pltpu.sync_copy(crows_a, su_hbm.at[pl.ds(ibase, piw)])
        cp_b.wait()
        pltpu.sync_copy(crows_b, si_hbm.at[pl.ds(ibase, piw)])

    return sc_gather


def _dense_body(nb_ref, ub_ref, ib_ref, bmem_ref, bout_ref, su_ref, si_ref,
                whop_ref, bhop_ref, wout_ref, bout2_ref, w1_ref, score_ref):
    d = whop_ref.shape[0]

    def half(packed, idx):
        sel = idx == 1
        return jnp.where(sel, packed[..., d:], packed[..., :d])

    mem = half(bmem_ref[...], nb_ref[...])
    nout = half(bout_ref[...], nb_ref[...])
    cu = half(su_ref[...], ub_ref[...])
    ci = half(si_ref[...], ib_ref[...])
    q = cu + ci

    def attend(query):
        s = jnp.sum(query[:, None, :] * mem, axis=2)
        s = s - jnp.max(s, axis=-1, keepdims=True)
        e = jnp.exp(s)
        a = e / jnp.sum(e, axis=-1, keepdims=True)
        return jnp.sum(a[:, :, None] * nout, axis=1)

    w0 = attend(q)
    q1 = jnp.dot(q, whop_ref[...], preferred_element_type=jnp.float32)
    q1 = jnp.maximum(q1 + bhop_ref[...] + w0, 0.0)
    w1 = attend(q1)

    aa = cu * ci
    z = (jnp.dot(aa, wout_ref[:d], preferred_element_type=jnp.float32)
         + jnp.dot(w1, wout_ref[d:], preferred_element_type=jnp.float32)
         + bout2_ref[...])
    score = jnp.maximum(jnp.sum(z * w1_ref[...], axis=1, keepdims=True), 0.0)
    score_ref[...] = score


def _dense(nb, ub, ib, bmem, bout, su, si, W_hop, b_hop, W_out, b_out, W_1,
           interpret=False):
    B = su.shape[0]
    L = bmem.shape[1]
    D = W_hop.shape[0]
    BB = 128
    grid = (B // BB,)
    return pl.pallas_call(
        _dense_body,
        grid=grid,
        in_specs=[
            pl.BlockSpec((BB, L, 1), lambda i: (i, 0, 0)),
            pl.BlockSpec((BB, 1), lambda i: (i, 0)),
            pl.BlockSpec((BB, 1), lambda i: (i, 0)),
            pl.BlockSpec((BB, L, 2 * D), lambda i: (i, 0, 0)),
            pl.BlockSpec((BB, L, 2 * D), lambda i: (i, 0, 0)),
            pl.BlockSpec((BB, 2 * D), lambda i: (i, 0)),
            pl.BlockSpec((BB, 2 * D), lambda i: (i, 0)),
            pl.BlockSpec((D, D), lambda i: (0, 0)),
            pl.BlockSpec((1, D), lambda i: (0, 0)),
            pl.BlockSpec((2 * D, D), lambda i: (0, 0)),
            pl.BlockSpec((1, D), lambda i: (0, 0)),
            pl.BlockSpec((1, D), lambda i: (0, 0)),
        ],
        out_specs=pl.BlockSpec((BB, 1), lambda i: (i, 0)),
        out_shape=jax.ShapeDtypeStruct((B, 1), jnp.float32),
        interpret=interpret,
    )(nb, ub, ib, bmem, bout, su, si, W_hop, b_hop, W_out, b_out, W_1)


def kernel(input_users, input_items, input_items_negative, input_neighborhoods,
           input_neighborhood_lengths, input_neighborhoods_negative,
           input_neighborhood_lengths_negative, user_memory, user_output,
           item_memory, W_hop, b_hop, W_out, b_out, W_1):
    B, L = input_neighborhoods.shape
    V, D = user_memory.shape

    nb = input_neighborhoods.astype(jnp.int32)
    ub = input_users.astype(jnp.int32).reshape(B, 1)
    ib = input_items.astype(jnp.int32).reshape(B, 1)
    hw = _TP_W // 2

    def pk(x):
        return (x // _TP_W) * hw + (x % hw)

    def sel(x):
        return (x // hw) & 1

    nb_sel, ub_sel, ib_sel = sel(nb), sel(ub), sel(ib)

    umem2, uout2, imem2 = _tc_transpose(
        user_memory.T, user_output.T, item_memory.T)

    sc_gather = _make_sc_gather(B * L, B, 2 * D)
    bmem, bout, su, si = sc_gather(
        pk(nb).reshape(-1), umem2, uout2,
        pk(ub).reshape(-1), pk(ib).reshape(-1),
        imem2)

    return _dense(nb_sel.reshape(B, L, 1), ub_sel, ib_sel,
                  bmem.reshape(B, L, 2 * D), bout.reshape(B, L, 2 * D),
                  su, si, W_hop,
                  b_hop.reshape(1, D), W_out, b_out.reshape(1, D),
                  W_1.reshape(1, D))
